# baseline (speedup 1.0000x reference)
import functools

import jax
import jax.numpy as jnp
from jax import lax
from jax.experimental import pallas as pl
from jax.experimental.pallas import tpu as pltpu
from jax.experimental.pallas import tpu_sc as plsc

_B = 4
_S = 4096
_D = 768
_LANES = 16
_NC = 2
_NS = 16
_NW = _NC * _NS
_SPW = _B * _S // _NW
_CHUNK = 16
_NCHUNK = _SPW // _CHUNK


def _body(inputs_hbm, table_hbm, out_hbm, ibuf, msk, buf):
    wid = lax.axis_index("s") * _NC + lax.axis_index("c")
    b = wid // (_NW // _B)
    s0 = (wid % (_NW // _B)) * _SPW

    def chunk(c, _):
        s = s0 + c * _CHUNK
        pltpu.sync_copy(table_hbm.at[pl.ds(s + 1, _CHUNK)], buf)
        pltpu.sync_copy(inputs_hbm.at[b, pl.ds(s, _CHUNK)], ibuf)
        ivec = ibuf[...]
        nz = jnp.sum(jnp.where(ivec == 0, 1, 0))

        @pl.when(nz > 0)
        def _fixup():
            msk[...] = jnp.where(ivec == 0, 0.0, 1.0)
            for r in range(_CHUNK):
                m = plsc.load_gather(
                    msk, [jnp.full((_LANES,), r, jnp.int32)])
                for j in range(_D // _LANES):
                    sl = pl.ds(j * _LANES, _LANES)
                    buf[r, sl] = buf[r, sl] * m

        pltpu.sync_copy(buf, out_hbm.at[b, pl.ds(s, _CHUNK)])
        return ()

    lax.fori_loop(0, _NCHUNK, chunk, ())


_mesh = plsc.VectorSubcoreMesh(
    core_axis_name="c", subcore_axis_name="s",
    num_cores=_NC, num_subcores=_NS)

_emb = functools.partial(
    pl.kernel,
    out_type=jax.ShapeDtypeStruct((_B, _S, _D), jnp.float32),
    mesh=_mesh,
    scratch_types=[
        pltpu.VMEM((_CHUNK,), jnp.int32),
        pltpu.VMEM((_LANES,), jnp.float32),
        pltpu.VMEM((_CHUNK, _D), jnp.float32),
    ],
    compiler_params=pltpu.CompilerParams(
        use_tc_tiling_on_sc=False, needs_layout_passes=False),
)(_body)


@jax.jit
def kernel(inputs, pos_weights):
    return _emb(inputs.astype(jnp.int32), pos_weights)


# speedup vs baseline: 1.4851x; 1.4851x over previous
import functools

import jax
import jax.numpy as jnp
from jax import lax
from jax.experimental import pallas as pl
from jax.experimental.pallas import tpu as pltpu
from jax.experimental.pallas import tpu_sc as plsc

_B = 4
_S = 4096
_D = 768
_LANES = 16
_NC = 2
_NS = 16
_NW = _NC * _NS
_SPW = _S // _NW
_G = _SPW // _LANES


def _body(inputs_hbm, table_hbm, out_hbm, ibuf, msk, tbuf, obuf, sem_t, sem_o):
    wid = lax.axis_index("s") * _NC + lax.axis_index("c")
    s0 = wid * _SPW

    tcopy = pltpu.async_copy(table_hbm.at[pl.ds(s0 + 1, _SPW)], tbuf, sem_t)
    pltpu.sync_copy(inputs_hbm.at[:, pl.ds(s0, _SPW)], ibuf)
    tcopy.wait()

    copies = [
        pltpu.async_copy(tbuf, out_hbm.at[b, pl.ds(s0, _SPW)], sem_o)
        for b in range(_B)
    ]
    for c in copies:
        c.wait()

    def fix_b(b, _):
        def fix_g(g, _):
            ivec = ibuf[b, pl.ds(g * _LANES, _LANES)]
            nz = jnp.sum(jnp.where(ivec == 0, 1, 0))

            @pl.when(nz > 0)
            def _fixup():
                msk[...] = jnp.where(ivec == 0, 0.0, 1.0)

                def fix_r(r, _):
                    m = plsc.load_gather(
                        msk, [jnp.full((_LANES,), 0, jnp.int32) + r])
                    row = g * _LANES + r
                    for j in range(_D // _LANES):
                        sl = pl.ds(j * _LANES, _LANES)
                        obuf[r, sl] = tbuf[row, sl] * m
                    return ()

                lax.fori_loop(0, _LANES, fix_r, ())
                pltpu.sync_copy(
                    obuf, out_hbm.at[b, pl.ds(s0 + g * _LANES, _LANES)])

            return ()

        lax.fori_loop(0, _G, fix_g, ())
        return ()

    lax.fori_loop(0, _B, fix_b, ())


_mesh = plsc.VectorSubcoreMesh(
    core_axis_name="c", subcore_axis_name="s",
    num_cores=_NC, num_subcores=_NS)

_emb = functools.partial(
    pl.kernel,
    out_type=jax.ShapeDtypeStruct((_B, _S, _D), jnp.float32),
    mesh=_mesh,
    scratch_types=[
        pltpu.VMEM((_B, _SPW), jnp.int32),
        pltpu.VMEM((_LANES,), jnp.float32),
        pltpu.VMEM((_SPW, _D), jnp.float32),
        pltpu.VMEM((_LANES, _D), jnp.float32),
        pltpu.SemaphoreType.DMA,
        pltpu.SemaphoreType.DMA,
    ],
    compiler_params=pltpu.CompilerParams(
        use_tc_tiling_on_sc=False, needs_layout_passes=False),
)(_body)


@jax.jit
def kernel(inputs, pos_weights):
    return _emb(inputs.astype(jnp.int32), pos_weights)


# speedup vs baseline: 4.3286x; 2.9147x over previous
import functools

import jax
import jax.numpy as jnp
from jax import lax
from jax.experimental import pallas as pl
from jax.experimental.pallas import tpu as pltpu
from jax.experimental.pallas import tpu_sc as plsc

_B = 4
_S = 4096
_D = 768
_LANES = 16
_NC = 2
_NS = 16
_NW = _NC * _NS
_SPW = _S // _NW
_G = _SPW // _LANES


def _body(inputs_hbm, table_hbm, out_hbm, ibuf, msk, idx, tbuf, obuf,
          sem_t, sem_o):
    wid = lax.axis_index("s") * _NC + lax.axis_index("c")
    s0 = wid * _SPW

    iot = lax.iota(jnp.int32, _LANES)
    for j in range(_SPW // _LANES):
        idx[pl.ds(j * _LANES, _LANES)] = iot + (s0 + 1 + j * _LANES)
    tcopy = pltpu.async_copy(table_hbm.at[idx], tbuf, sem_t)
    pltpu.sync_copy(inputs_hbm.at[:, pl.ds(s0, _SPW)], ibuf)
    tcopy.wait()

    copies = [
        pltpu.async_copy(tbuf, out_hbm.at[b, pl.ds(s0, _SPW)], sem_o)
        for b in range(_B)
    ]
    for c in copies:
        c.wait()

    def fix_b(b, _):
        def fix_g(g, _):
            ivec = ibuf[b, pl.ds(g * _LANES, _LANES)]
            nz = jnp.sum(jnp.where(ivec == 0, 1, 0))

            @pl.when(nz > 0)
            def _fixup():
                msk[...] = jnp.where(ivec == 0, 0.0, 1.0)

                def fix_r(r, _):
                    m = plsc.load_gather(
                        msk, [jnp.full((_LANES,), 0, jnp.int32) + r])
                    row = g * _LANES + r
                    for j in range(_D // _LANES):
                        sl = pl.ds(j * _LANES, _LANES)
                        obuf[r, sl] = tbuf[row, sl] * m
                    return ()

                lax.fori_loop(0, _LANES, fix_r, ())
                pltpu.sync_copy(
                    obuf, out_hbm.at[b, pl.ds(s0 + g * _LANES, _LANES)])

            return ()

        lax.fori_loop(0, _G, fix_g, ())
        return ()

    lax.fori_loop(0, _B, fix_b, ())


_mesh = plsc.VectorSubcoreMesh(
    core_axis_name="c", subcore_axis_name="s",
    num_cores=_NC, num_subcores=_NS)

_emb = functools.partial(
    pl.kernel,
    out_type=jax.ShapeDtypeStruct((_B, _S, _D), jnp.float32),
    mesh=_mesh,
    scratch_types=[
        pltpu.VMEM((_B, _SPW), jnp.int32),
        pltpu.VMEM((_LANES,), jnp.float32),
        pltpu.VMEM((_SPW,), jnp.int32),
        pltpu.VMEM((_SPW, _D), jnp.float32),
        pltpu.VMEM((_LANES, _D), jnp.float32),
        pltpu.SemaphoreType.DMA,
        pltpu.SemaphoreType.DMA,
    ],
    compiler_params=pltpu.CompilerParams(needs_layout_passes=False),
)(_body)


@jax.jit
def kernel(inputs, pos_weights):
    return _emb(inputs.astype(jnp.int32), pos_weights)
